# baseline (speedup 1.0000x reference)
import jax
import jax.numpy as jnp
from jax.experimental import pallas as pl
from jax.experimental.pallas import tpu as pltpu

_N = 10000
_M = 2500
_K = 32
_K_UP = 3
_STRIDE = 4
_D = 128


def _shift(x, s):
    return jnp.concatenate([x[s:], x[:s]], axis=0)


def _win_sum_pow2(x, w):
    s = x
    span = 1
    while span < w:
        s = s + _shift(s, span)
        span *= 2
    return s


def _body(featr_ref, wd_ref, bd_ref, wf_ref, bf_ref, ws_ref, bs_ref,
          wma_ref, wmb_ref, bm_ref, wu_ref, wua_ref, wub_ref, bu_ref,
          out_ref):
    f32 = jnp.float32

    def mm(a, b):
        return jax.lax.dot_general(a.astype(jnp.bfloat16),
                                   b.astype(jnp.bfloat16),
                                   (((1,), (0,)), ((), ())),
                                   preferred_element_type=f32)

    bd = bd_ref[0, :]
    c = None
    for j in range(_STRIDE):
        fj = featr_ref[:, j * _D:(j + 1) * _D]
        yj = jnp.maximum(mm(fj, wd_ref[...]) + bd, 0.0)
        c = yj if c is None else c + yj
    feat1 = _win_sum_pow2(c, _K // _STRIDE) * (1.0 / _K)

    y1 = jnp.maximum(mm(feat1, wf_ref[...]) + bf_ref[0, :], 0.0)
    feat2 = _win_sum_pow2(y1, _K) * (1.0 / _K)

    ys = jnp.maximum(mm(feat2, ws_ref[...]) + bs_ref[0, :], 0.0)
    skip = _win_sum_pow2(ys, _K) * (1.0 / _K)

    ym = jnp.maximum(mm(feat2, wma_ref[...]) + mm(skip, wmb_ref[...])
                     + bm_ref[0, :], 0.0)
    merge = _win_sum_pow2(ym, _K) * (1.0 / _K)

    z = (mm(merge, wu_ref[...]) + mm(feat2, wua_ref[...])
         + mm(skip, wub_ref[...]) + bu_ref[0, :])
    u = (z + _shift(z, 1) + _shift(z, 2)) * (1.0 / _K_UP)

    out_ref[...] = jnp.concatenate([u, u, u, u], axis=1)


def kernel(point_bcenter, point_feat, W_d0, b_d0, W_f0, b_f0, W_s, b_s,
           W_m, b_m, W_u, b_u):
    del point_bcenter
    featr = point_feat.reshape(_M, _STRIDE * _D)
    w_u_rep = jnp.repeat(W_u, 2, axis=0)
    args = (
        featr,
        W_d0, b_d0.reshape(1, _D),
        W_f0, b_f0.reshape(1, _D),
        W_s, b_s.reshape(1, _D),
        W_m[:_D], W_m[_D:], b_m.reshape(1, _D),
        W_u, w_u_rep[:_D], w_u_rep[_D:], b_u.reshape(1, _D),
    )
    out = pl.pallas_call(
        _body,
        out_shape=jax.ShapeDtypeStruct((_M, _STRIDE * _D), jnp.float32),
        compiler_params=pltpu.CompilerParams(
            vmem_limit_bytes=100 * 1024 * 1024),
    )(*args)
    return out.reshape(_N, _D)


# speedup vs baseline: 1.0025x; 1.0025x over previous
import jax
import jax.numpy as jnp
from jax.experimental import pallas as pl
from jax.experimental.pallas import tpu as pltpu

_N = 10000
_M = 2500
_K = 32
_K_UP = 3
_STRIDE = 4
_D = 128


def _shift(x, s):
    return pltpu.roll(x, x.shape[0] - s, 0)


def _win_sum_pow2(x, w):
    s = x
    span = 1
    while span < w:
        s = s + _shift(s, span)
        span *= 2
    return s


def _body(featr_ref, wd_ref, bd_ref, wf_ref, bf_ref, ws_ref, bs_ref,
          wma_ref, wmb_ref, bm_ref, wu_ref, wua_ref, wub_ref, bu_ref,
          out_ref):
    f32 = jnp.float32

    def mm(a, b):
        return jax.lax.dot_general(a.astype(jnp.bfloat16),
                                   b.astype(jnp.bfloat16),
                                   (((1,), (0,)), ((), ())),
                                   preferred_element_type=f32)

    bd = bd_ref[0, :]
    c = None
    for j in range(_STRIDE):
        fj = featr_ref[:, j * _D:(j + 1) * _D]
        yj = jnp.maximum(mm(fj, wd_ref[...]) + bd, 0.0)
        c = yj if c is None else c + yj
    feat1 = _win_sum_pow2(c, _K // _STRIDE) * (1.0 / _K)

    y1 = jnp.maximum(mm(feat1, wf_ref[...]) + bf_ref[0, :], 0.0)
    feat2 = _win_sum_pow2(y1, _K) * (1.0 / _K)

    ys = jnp.maximum(mm(feat2, ws_ref[...]) + bs_ref[0, :], 0.0)
    skip = _win_sum_pow2(ys, _K) * (1.0 / _K)

    ym = jnp.maximum(mm(feat2, wma_ref[...]) + mm(skip, wmb_ref[...])
                     + bm_ref[0, :], 0.0)
    merge = _win_sum_pow2(ym, _K) * (1.0 / _K)

    z = (mm(merge, wu_ref[...]) + mm(feat2, wua_ref[...])
         + mm(skip, wub_ref[...]) + bu_ref[0, :])
    u = (z + _shift(z, 1) + _shift(z, 2)) * (1.0 / _K_UP)

    out_ref[...] = jnp.concatenate([u, u, u, u], axis=1)


def kernel(point_bcenter, point_feat, W_d0, b_d0, W_f0, b_f0, W_s, b_s,
           W_m, b_m, W_u, b_u):
    del point_bcenter
    featr = point_feat.reshape(_M, _STRIDE * _D)
    w_u_rep = jnp.repeat(W_u, 2, axis=0)
    args = (
        featr,
        W_d0, b_d0.reshape(1, _D),
        W_f0, b_f0.reshape(1, _D),
        W_s, b_s.reshape(1, _D),
        W_m[:_D], W_m[_D:], b_m.reshape(1, _D),
        W_u, w_u_rep[:_D], w_u_rep[_D:], b_u.reshape(1, _D),
    )
    out = pl.pallas_call(
        _body,
        out_shape=jax.ShapeDtypeStruct((_M, _STRIDE * _D), jnp.float32),
        compiler_params=pltpu.CompilerParams(
            vmem_limit_bytes=100 * 1024 * 1024),
    )(*args)
    return out.reshape(_N, _D)
